# baseline (speedup 1.0000x reference)
import functools

import jax
import jax.numpy as jnp
from jax import lax
from jax.experimental import pallas as pl
from jax.experimental.pallas import tpu as pltpu
from jax.experimental.pallas import tpu_sc as plsc

N = 10000
D = 256
E = 160000
HALF = 128
NCORE = 2
NSUB = 16
CHUNK = 128
NB = 80
E_PAD = NSUB * NB * CHUNK
NRING = 3
ROWS_ACC = N + 16
RPW = 624
TAIL = N - NSUB * RPW


def _sc_aggregate(x, packed):
    mesh = plsc.VectorSubcoreMesh(core_axis_name="c", subcore_axis_name="s")

    @functools.partial(
        pl.kernel,
        mesh=mesh,
        out_type=jax.ShapeDtypeStruct((NCORE, N, HALF), jnp.float32),
        scratch_types=[
            pltpu.VMEM((8, CHUNK), jnp.int32),
            pltpu.VMEM((NRING, CHUNK, HALF), jnp.float32),
            pltpu.VMEM_SHARED((ROWS_ACC, HALF), jnp.float32),
        ] + [pltpu.SemaphoreType.DMA] * (2 + 2 * NRING),
    )
    def k(packed_hbm, x_hbm, out_hbm, su, rows_v, acc, *sems):
        c = lax.axis_index("c")
        s = lax.axis_index("s")
        xh = x_hbm.at[:, pl.ds(c * HALF, HALF)]
        pksems = sems[:2]
        gsems = sems[2:2 + NRING]
        ssems = sems[2 + NRING:]
        PKROW = (3, 7)

        def zrow(i, carry):
            def zcol(j, carry2):
                rows_v[0, i, pl.ds(j * 16, 16)] = jnp.zeros((16,), jnp.float32)
                return carry2
            return lax.fori_loop(0, HALF // 16, zcol, carry)
        lax.fori_loop(0, CHUNK, zrow, 0)
        zslab = rows_v.at[0]
        for t in range(RPW // CHUNK):
            pltpu.sync_copy(zslab, acc.at[pl.ds(s * RPW + t * CHUNK, CHUNK), :])
        rem = RPW - (RPW // CHUNK) * CHUNK
        if rem:
            pltpu.sync_copy(zslab.at[pl.ds(0, rem), :],
                            acc.at[pl.ds(s * RPW + RPW - rem, rem), :])

        @pl.when(s == NSUB - 1)
        def _zero_tail():
            pltpu.sync_copy(zslab.at[pl.ds(0, TAIL), :],
                            acc.at[pl.ds(NSUB * RPW, TAIL), :])
        plsc.subcore_barrier()

        def start_pk(j, q):
            pltpu.async_copy(packed_hbm.at[s, j], su.at[PKROW[q]], pksems[q])

        def start_gather(j, r, q):
            pltpu.make_async_copy(packed_hbm.at[s, j], su.at[PKROW[q]],
                                  pksems[q]).wait()
            for t in range(CHUNK // 16):
                p = su[PKROW[q], pl.ds(t * 16, 16)]
                su[r, pl.ds(t * 16, 16)] = jnp.right_shift(p, 14)
                su[4 + r, pl.ds(t * 16, 16)] = jnp.bitwise_and(p, 16383)
            pltpu.async_copy(xh.at[su.at[r]], rows_v.at[r], gsems[r])

        def start_scatter(r):
            pltpu.make_async_copy(xh.at[su.at[r]], rows_v.at[r],
                                  gsems[r]).wait()
            pltpu.async_copy(rows_v.at[r], acc.at[su.at[4 + r]], ssems[r],
                             add=True)

        def wait_scatter(r):
            pltpu.make_async_copy(rows_v.at[r], acc.at[su.at[4 + r]],
                                  ssems[r]).wait()

        start_pk(0, 0)
        start_pk(1, 1)
        start_gather(0, 0, 0)
        start_pk(2, 0)
        start_gather(1, 1, 1)
        start_pk(3, 1)

        def body(t, carry):
            j = 6 * t
            for bb in range(6):
                jj = j + bb
                r = bb % 3
                q = bb % 2

                @pl.when(jnp.logical_and(jj >= 1, jj <= NB))
                def _drain():
                    wait_scatter((r + 2) % 3)

                @pl.when(jj + 2 < NB)
                def _gather():
                    start_gather(jj + 2, (r + 2) % 3, q)

                @pl.when(jj < NB)
                def _this():
                    start_scatter(r)

                @pl.when(jj + 4 < NB)
                def _pk():
                    start_pk(jj + 4, q)
            return carry
        lax.fori_loop(0, (NB + 6) // 6, body, 0)
        plsc.subcore_barrier()

        pltpu.sync_copy(acc.at[pl.ds(s * RPW, RPW), :],
                        out_hbm.at[c, pl.ds(s * RPW, RPW)])

        @pl.when(s == NSUB - 1)
        def _write_tail():
            pltpu.sync_copy(acc.at[pl.ds(NSUB * RPW, TAIL), :],
                            out_hbm.at[c, pl.ds(NSUB * RPW, TAIL)])

    return k(packed, x)


def _tc_dense(x, aggr2, W, b, eps):
    R = 1000

    def body(eps_ref, x_ref, a_ref, w_ref, b_ref, o_ref):
        e1 = 1.0 + eps_ref[0, 0]
        w = w_ref[...]
        h0 = e1 * x_ref[:, :HALF] + a_ref[0]
        h1 = e1 * x_ref[:, HALF:] + a_ref[1]
        acc = lax.dot_general(h0, w[:, :HALF], (((1,), (1,)), ((), ())),
                              preferred_element_type=jnp.float32)
        acc = acc + lax.dot_general(h1, w[:, HALF:], (((1,), (1,)), ((), ())),
                                    preferred_element_type=jnp.float32)
        o_ref[...] = jnp.maximum(acc + b_ref[...], 0.0)

    return pl.pallas_call(
        body,
        grid=(N // R,),
        in_specs=[
            pl.BlockSpec(memory_space=pltpu.SMEM),
            pl.BlockSpec((R, D), lambda i: (i, 0)),
            pl.BlockSpec((NCORE, R, HALF), lambda i: (0, i, 0)),
            pl.BlockSpec((D, D), lambda i: (0, 0)),
            pl.BlockSpec((1, D), lambda i: (0, 0)),
        ],
        out_specs=pl.BlockSpec((R, D), lambda i: (i, 0)),
        out_shape=jax.ShapeDtypeStruct((N, D), jnp.float32),
    )(eps.reshape(1, 1).astype(jnp.float32), x, aggr2, W, b.reshape(1, D))


def kernel(x, edge_index, W, b, eps):
    src = edge_index[0]
    dst = edge_index[1]
    pad = E_PAD - E
    pad_src = jnp.arange(pad, dtype=jnp.int32) % jnp.int32(N)
    pad_dst = jnp.int32(N) + jnp.arange(pad, dtype=jnp.int32) % jnp.int32(16)
    srcp = jnp.concatenate([src, pad_src])
    dstp = jnp.concatenate([dst, pad_dst])
    packed = (srcp * 16384 + dstp).reshape(NSUB, NB, CHUNK)
    aggr2 = _sc_aggregate(x, packed)
    return _tc_dense(x, aggr2, W, b, eps)


# speedup vs baseline: 1.0011x; 1.0011x over previous
import functools

import jax
import jax.numpy as jnp
from jax import lax
from jax.experimental import pallas as pl
from jax.experimental.pallas import tpu as pltpu
from jax.experimental.pallas import tpu_sc as plsc

N = 10000
D = 256
E = 160000
HALF = 128
NCORE = 2
NSUB = 16
CHUNK = 128
NB = 80
E_PAD = NSUB * NB * CHUNK
NRING = 3
ROWS_ACC = N + 16
RPW = 624
TAIL = N - NSUB * RPW


def _sc_aggregate(x, packed):
    mesh = plsc.VectorSubcoreMesh(core_axis_name="c", subcore_axis_name="s")

    @functools.partial(
        pl.kernel,
        mesh=mesh,
        out_type=jax.ShapeDtypeStruct((NCORE, N, HALF), jnp.float32),
        scratch_types=[
            pltpu.VMEM((8, CHUNK), jnp.int32),
            pltpu.VMEM((NRING, CHUNK, HALF), jnp.float32),
            pltpu.VMEM_SHARED((ROWS_ACC, HALF), jnp.float32),
        ] + [pltpu.SemaphoreType.DMA] * (2 + 2 * NRING),
    )
    def k(packed_hbm, x_hbm, out_hbm, su, rows_v, acc, *sems):
        c = lax.axis_index("c")
        s = lax.axis_index("s")
        xh = x_hbm.at[:, pl.ds(c * HALF, HALF)]
        pksems = sems[:2]
        gsems = sems[2:2 + NRING]
        ssems = sems[2 + NRING:]
        PKROW = (3, 7)

        def start_pk(j, q):
            pltpu.async_copy(packed_hbm.at[s, j], su.at[PKROW[q]], pksems[q])

        def start_gather(j, r, q):
            pltpu.make_async_copy(packed_hbm.at[s, j], su.at[PKROW[q]],
                                  pksems[q]).wait()
            for t in range(CHUNK // 16):
                p = su[PKROW[q], pl.ds(t * 16, 16)]
                su[r, pl.ds(t * 16, 16)] = jnp.right_shift(p, 14)
                su[4 + r, pl.ds(t * 16, 16)] = jnp.bitwise_and(p, 16383)
            pltpu.async_copy(xh.at[su.at[r]], rows_v.at[r], gsems[r])

        def start_scatter(r):
            pltpu.make_async_copy(xh.at[su.at[r]], rows_v.at[r],
                                  gsems[r]).wait()
            pltpu.async_copy(rows_v.at[r], acc.at[su.at[4 + r]], ssems[r],
                             add=True)

        def wait_scatter(r):
            pltpu.make_async_copy(rows_v.at[r], acc.at[su.at[4 + r]],
                                  ssems[r]).wait()

        start_pk(0, 0)
        start_pk(1, 1)
        start_gather(0, 0, 0)
        start_pk(2, 0)
        start_gather(1, 1, 1)
        start_pk(3, 1)

        def zrow(i, carry):
            def zcol(j, carry2):
                rows_v[2, i, pl.ds(j * 16, 16)] = jnp.zeros((16,), jnp.float32)
                return carry2
            return lax.fori_loop(0, HALF // 16, zcol, carry)
        lax.fori_loop(0, CHUNK, zrow, 0)
        zslab = rows_v.at[2]
        for t in range(RPW // CHUNK):
            pltpu.sync_copy(zslab, acc.at[pl.ds(s * RPW + t * CHUNK, CHUNK), :])
        rem = RPW - (RPW // CHUNK) * CHUNK
        if rem:
            pltpu.sync_copy(zslab.at[pl.ds(0, rem), :],
                            acc.at[pl.ds(s * RPW + RPW - rem, rem), :])

        @pl.when(s == NSUB - 1)
        def _zero_tail():
            pltpu.sync_copy(zslab.at[pl.ds(0, TAIL), :],
                            acc.at[pl.ds(NSUB * RPW, TAIL), :])
        plsc.subcore_barrier()

        def body(t, carry):
            j = 6 * t
            for bb in range(6):
                jj = j + bb
                r = bb % 3
                q = bb % 2

                @pl.when(jnp.logical_and(jj >= 1, jj <= NB))
                def _drain():
                    wait_scatter((r + 2) % 3)

                @pl.when(jj + 2 < NB)
                def _gather():
                    start_gather(jj + 2, (r + 2) % 3, q)

                @pl.when(jj < NB)
                def _this():
                    start_scatter(r)

                @pl.when(jj + 4 < NB)
                def _pk():
                    start_pk(jj + 4, q)
            return carry
        lax.fori_loop(0, (NB + 6) // 6, body, 0)
        plsc.subcore_barrier()

        pltpu.sync_copy(acc.at[pl.ds(s * RPW, RPW), :],
                        out_hbm.at[c, pl.ds(s * RPW, RPW)])

        @pl.when(s == NSUB - 1)
        def _write_tail():
            pltpu.sync_copy(acc.at[pl.ds(NSUB * RPW, TAIL), :],
                            out_hbm.at[c, pl.ds(NSUB * RPW, TAIL)])

    return k(packed, x)


def _tc_dense(x, aggr2, W, b, eps):
    R = 1000

    def body(eps_ref, x_ref, a_ref, w_ref, b_ref, o_ref):
        e1 = 1.0 + eps_ref[0, 0]
        w = w_ref[...]
        h0 = e1 * x_ref[:, :HALF] + a_ref[0]
        h1 = e1 * x_ref[:, HALF:] + a_ref[1]
        acc = lax.dot_general(h0, w[:, :HALF], (((1,), (1,)), ((), ())),
                              preferred_element_type=jnp.float32)
        acc = acc + lax.dot_general(h1, w[:, HALF:], (((1,), (1,)), ((), ())),
                                    preferred_element_type=jnp.float32)
        o_ref[...] = jnp.maximum(acc + b_ref[...], 0.0)

    return pl.pallas_call(
        body,
        grid=(N // R,),
        in_specs=[
            pl.BlockSpec(memory_space=pltpu.SMEM),
            pl.BlockSpec((R, D), lambda i: (i, 0)),
            pl.BlockSpec((NCORE, R, HALF), lambda i: (0, i, 0)),
            pl.BlockSpec((D, D), lambda i: (0, 0)),
            pl.BlockSpec((1, D), lambda i: (0, 0)),
        ],
        out_specs=pl.BlockSpec((R, D), lambda i: (i, 0)),
        out_shape=jax.ShapeDtypeStruct((N, D), jnp.float32),
    )(eps.reshape(1, 1).astype(jnp.float32), x, aggr2, W, b.reshape(1, D))


def kernel(x, edge_index, W, b, eps):
    src = edge_index[0]
    dst = edge_index[1]
    pad = E_PAD - E
    pad_src = jnp.arange(pad, dtype=jnp.int32) % jnp.int32(N)
    pad_dst = jnp.int32(N) + jnp.arange(pad, dtype=jnp.int32) % jnp.int32(16)
    srcp = jnp.concatenate([src, pad_src])
    dstp = jnp.concatenate([dst, pad_dst])
    packed = (srcp * 16384 + dstp).reshape(NSUB, NB, CHUNK)
    aggr2 = _sc_aggregate(x, packed)
    return _tc_dense(x, aggr2, W, b, eps)
